# baseline (device time: 21966 ns/iter reference)
import jax
import jax.numpy as jnp
from jax import lax
from jax.experimental import pallas as pl
from jax.experimental.pallas import tpu as pltpu

N_DEV = 16
B, SQ, SKV = 2, 128, 128
H_LOC, DH = 4, 64
CHUNK = H_LOC * DH
ROWS = B * SQ
D_OUT = 512
PIECE = ROWS // N_DEV


def _body(x_ref, wq_ref, k_ref, v_ref, wo_ref, out_ref,
          ctx_buf, stage, rs_buf, ag_buf,
          rs_send_sems, rs_recv_sems, ag_send_sems, ag_recv_sems):
    my = lax.axis_index("i")

    barrier = pltpu.get_barrier_semaphore()
    for k in range(1, N_DEV):
        pl.semaphore_signal(barrier, inc=1,
                            device_id=(lax.rem(my + k, N_DEV),),
                            device_id_type=pl.DeviceIdType.MESH)

    xb = x_ref[...].astype(jnp.bfloat16)
    wqb = wq_ref[...].astype(jnp.bfloat16)
    q2d = jax.lax.dot_general(xb, wqb, (((1,), (0,)), ((), ())),
                              preferred_element_type=jnp.float32)
    q2d = q2d.astype(jnp.bfloat16)

    qi = jax.lax.broadcasted_iota(jnp.int32, (SQ, SKV), 0)
    kj = jax.lax.broadcasted_iota(jnp.int32, (SQ, SKV), 1)
    qb_, kb_ = qi // 64, kj // 64
    mask = (qb_ == kb_) | ((kb_ % 4) == (qb_ % 4))

    for b in range(B):
        for h in range(H_LOC):
            qbh = q2d[b * SQ:(b + 1) * SQ, h * DH:(h + 1) * DH]
            kbh = k_ref[b, :, h, :].astype(jnp.bfloat16)
            vbh = v_ref[b, :, h, :].astype(jnp.bfloat16)
            s = jax.lax.dot_general(qbh, kbh, (((1,), (1,)), ((), ())),
                                    preferred_element_type=jnp.float32)
            s = s * 0.125
            s = jnp.where(mask, s, -1e9)
            s = s - jnp.max(s, axis=1, keepdims=True)
            e = jnp.exp(s)
            w = (e / jnp.sum(e, axis=1, keepdims=True)).astype(jnp.bfloat16)
            c = jax.lax.dot_general(w, vbh, (((1,), (0,)), ((), ())),
                                    preferred_element_type=jnp.float32)
            ctx_buf[b * SQ:(b + 1) * SQ,
                    h * DH:(h + 1) * DH] = c.astype(jnp.bfloat16)

    wob = wo_ref[...].astype(jnp.bfloat16)
    partial = jax.lax.dot_general(ctx_buf[...], wob, (((1,), (0,)), ((), ())),
                                  preferred_element_type=jnp.float32)

    stage[...] = partial.astype(jnp.bfloat16).reshape(N_DEV, PIECE, D_OUT)
    rs_buf[pl.ds(my, 1)] = stage[pl.ds(my, 1)]

    pl.semaphore_wait(barrier, N_DEV - 1)

    rs_rdmas = []
    for k in range(1, N_DEV):
        j = lax.rem(my + k, N_DEV)
        rdma = pltpu.make_async_remote_copy(
            src_ref=stage.at[j],
            dst_ref=rs_buf.at[my],
            send_sem=rs_send_sems.at[j],
            recv_sem=rs_recv_sems.at[my],
            device_id=(j,),
            device_id_type=pl.DeviceIdType.MESH,
        )
        rdma.start()
        rs_rdmas.append(rdma)
    for k in range(1, N_DEV):
        s = lax.rem(my + k, N_DEV)
        recv = pltpu.make_async_remote_copy(
            src_ref=stage.at[s],
            dst_ref=rs_buf.at[s],
            send_sem=rs_send_sems.at[s],
            recv_sem=rs_recv_sems.at[s],
            device_id=(s,),
            device_id_type=pl.DeviceIdType.MESH,
        )
        recv.wait_recv()

    piece = jnp.sum(rs_buf[...].astype(jnp.float32), axis=0)

    ag_buf[pl.ds(my, 1)] = piece.astype(jnp.bfloat16)[None]
    ag_rdmas = []
    for k in range(1, N_DEV):
        j = lax.rem(my + k, N_DEV)
        rdma = pltpu.make_async_remote_copy(
            src_ref=ag_buf.at[my],
            dst_ref=ag_buf.at[my],
            send_sem=ag_send_sems.at[j],
            recv_sem=ag_recv_sems.at[my],
            device_id=(j,),
            device_id_type=pl.DeviceIdType.MESH,
        )
        rdma.start()
        ag_rdmas.append(rdma)
    for k in range(1, N_DEV):
        s = lax.rem(my + k, N_DEV)
        recv = pltpu.make_async_remote_copy(
            src_ref=ag_buf.at[s],
            dst_ref=ag_buf.at[s],
            send_sem=ag_send_sems.at[s],
            recv_sem=ag_recv_sems.at[s],
            device_id=(s,),
            device_id_type=pl.DeviceIdType.MESH,
        )
        recv.wait_recv()

    out_ref[...] = ag_buf[...].astype(jnp.float32).reshape(ROWS, D_OUT)

    for rdma in rs_rdmas + ag_rdmas:
        rdma.wait_send()


def kernel(x, Wq, K_ext, V_ext, Wo):
    my = lax.axis_index("i")

    x2d = x.reshape(ROWS, x.shape[2])
    Wq_loc = lax.dynamic_slice(Wq, (0, my * CHUNK), (Wq.shape[0], CHUNK))
    Wo_loc = lax.dynamic_slice(Wo, (my * CHUNK, 0), (CHUNK, Wo.shape[1]))

    out = pl.pallas_call(
        _body,
        out_shape=jax.ShapeDtypeStruct((ROWS, D_OUT), jnp.float32),
        in_specs=[pl.BlockSpec(memory_space=pltpu.VMEM)] * 5,
        out_specs=pl.BlockSpec(memory_space=pltpu.VMEM),
        scratch_shapes=[
            pltpu.VMEM((ROWS, CHUNK), jnp.bfloat16),
            pltpu.VMEM((N_DEV, PIECE, D_OUT), jnp.bfloat16),
            pltpu.VMEM((N_DEV, PIECE, D_OUT), jnp.bfloat16),
            pltpu.VMEM((N_DEV, PIECE, D_OUT), jnp.bfloat16),
            pltpu.SemaphoreType.DMA((N_DEV,)),
            pltpu.SemaphoreType.DMA((N_DEV,)),
            pltpu.SemaphoreType.DMA((N_DEV,)),
            pltpu.SemaphoreType.DMA((N_DEV,)),
        ],
        compiler_params=pltpu.CompilerParams(collective_id=0),
    )(x2d, Wq_loc, K_ext, V_ext, Wo_loc)
    return out.reshape(B, SQ, D_OUT)
